# baseline (speedup 1.0000x reference)
import dataclasses
import functools

import jax
import jax.numpy as jnp
from jax import lax
from jax.experimental import pallas as pl
from jax.experimental.pallas import tpu as pltpu
from jax.experimental.pallas import tpu_sc as plsc

NC = 2
NS = 16
L = 16
NW = NC * NS
B = 4096
H = 200
D = 64
BPW = B // NW
GW = 100
NG = H // GW


def _sc_compiler_params():
    cp = pltpu.CompilerParams()
    fields = pltpu.CompilerParams.__dataclass_fields__
    if "needs_layout_passes" in fields:
        cp = dataclasses.replace(cp, needs_layout_passes=False)
    if "use_tc_tiling_on_sc" in fields:
        cp = dataclasses.replace(cp, use_tc_tiling_on_sc=False)
    return cp


def _sc_embedding_bag(ctx3, ctx_v, itm_flat, embed_w):
    mesh = plsc.VectorSubcoreMesh(core_axis_name="c", subcore_axis_name="s")

    @functools.partial(
        pl.kernel,
        out_type=[jax.ShapeDtypeStruct((B, D), jnp.float32),
                  jax.ShapeDtypeStruct((B, D), jnp.float32)],
        mesh=mesh,
        compiler_params=_sc_compiler_params(),
        scratch_types=[
            pltpu.VMEM((BPW, NG, GW), jnp.int32),
            pltpu.VMEM((BPW, H), jnp.float32),
            pltpu.VMEM((H, D), jnp.float32),
            pltpu.VMEM((BPW, D), jnp.float32),
            pltpu.VMEM((BPW,), jnp.int32),
            pltpu.VMEM((BPW, D), jnp.float32),
        ],
    )
    def k(ctx_hbm, ctxv_hbm, itm_hbm, tab_hbm, ctxsum_hbm, itmrows_hbm,
          idx_v, w_v, rows_v, out_v, itmidx_v, itmrows_v):
        wid = lax.axis_index("s") * NC + lax.axis_index("c")
        base = wid * BPW

        pltpu.sync_copy(itm_hbm.at[pl.ds(base, BPW)], itmidx_v)
        pltpu.sync_copy(tab_hbm.at[itmidx_v], itmrows_v)
        pltpu.sync_copy(itmrows_v, itmrows_hbm.at[pl.ds(base, BPW)])

        pltpu.sync_copy(ctx_hbm.at[pl.ds(base, BPW)], idx_v)
        pltpu.sync_copy(ctxv_hbm.at[pl.ds(base, BPW)], w_v)

        @pl.loop(0, BPW)
        def _(i):
            pltpu.sync_copy(tab_hbm.at[idx_v.at[i, 0]], rows_v.at[pl.ds(0, GW)])
            pltpu.sync_copy(tab_hbm.at[idx_v.at[i, 1]], rows_v.at[pl.ds(GW, GW)])

            def body(l, accs):
                wv = plsc.load_gather(
                    w_v, [jnp.full((L,), i, jnp.int32),
                          jnp.full((L,), l, jnp.int32)])
                return tuple(acc + wv * rows_v[l, pl.ds(j * L, L)]
                             for j, acc in enumerate(accs))

            accs = lax.fori_loop(
                0, H, body,
                tuple(jnp.zeros((L,), jnp.float32) for _ in range(D // L)))
            for j in range(D // L):
                out_v[i, pl.ds(j * L, L)] = accs[j]

        pltpu.sync_copy(out_v, ctxsum_hbm.at[pl.ds(base, BPW)])

    return k(ctx3, ctx_v, itm_flat, embed_w)


def _tc_tail(ctx_sum, itm_rows, z, fc1_w, fc1_b):
    def body(cs_ref, it_ref, z_ref, w_ref, b_ref, o_ref):
        p = cs_ref[...] * it_ref[...] * w_ref[:, :D] + z_ref[...] * w_ref[:, D:]
        o_ref[...] = jnp.sum(p, axis=1, keepdims=True) + b_ref[...]

    return pl.pallas_call(
        body,
        out_shape=jax.ShapeDtypeStruct((B, 1), jnp.float32),
    )(ctx_sum, itm_rows, z, fc1_w, fc1_b)


def kernel(ctx, itm, pos, ctx_v, z, embed_w, fc1_w, fc1_b):
    del pos
    ctx3 = ctx.reshape(B, NG, GW)
    itm_flat = itm.reshape(B)
    ctx_sum, itm_rows = _sc_embedding_bag(ctx3, ctx_v, itm_flat, embed_w)
    return _tc_tail(ctx_sum, itm_rows, z, fc1_w, fc1_b.reshape(1, 1))


# speedup vs baseline: 1.2676x; 1.2676x over previous
import dataclasses
import functools

import jax
import jax.numpy as jnp
from jax import lax
from jax.experimental import pallas as pl
from jax.experimental.pallas import tpu as pltpu
from jax.experimental.pallas import tpu_sc as plsc

NC = 2
NS = 16
L = 16
NW = NC * NS
B = 4096
H = 200
D = 64
BPW = B // NW
G0 = 128
G1 = H - G0
UNROLL = 8


def _sc_compiler_params():
    cp = pltpu.CompilerParams()
    fields = pltpu.CompilerParams.__dataclass_fields__
    if "needs_layout_passes" in fields:
        cp = dataclasses.replace(cp, needs_layout_passes=False)
    if "use_tc_tiling_on_sc" in fields:
        cp = dataclasses.replace(cp, use_tc_tiling_on_sc=False)
    return cp


def _sc_embedding_bag(ctx, ctx_v, itm_flat, embed_w):
    mesh = plsc.VectorSubcoreMesh(core_axis_name="c", subcore_axis_name="s")

    @functools.partial(
        pl.kernel,
        out_type=[jax.ShapeDtypeStruct((B, D), jnp.float32),
                  jax.ShapeDtypeStruct((B, D), jnp.float32)],
        mesh=mesh,
        compiler_params=_sc_compiler_params(),
        scratch_types=[
            pltpu.VMEM((BPW, H), jnp.int32),
            pltpu.VMEM((BPW, H), jnp.float32),
            pltpu.VMEM((H, D), jnp.float32),
            pltpu.VMEM((H, D), jnp.float32),
            pltpu.VMEM((BPW, D), jnp.float32),
            pltpu.VMEM((BPW,), jnp.int32),
            pltpu.VMEM((BPW, D), jnp.float32),
            pltpu.SemaphoreType.DMA,
            pltpu.SemaphoreType.DMA,
        ],
    )
    def k(ctx_hbm, ctxv_hbm, itm_hbm, tab_hbm, ctxsum_hbm, itmrows_hbm,
          idx_v, w_v, rows0, rows1, out_v, itmidx_v, itmrows_v, sem0, sem1):
        wid = lax.axis_index("s") * NC + lax.axis_index("c")
        base = wid * BPW

        pltpu.sync_copy(ctx_hbm.at[pl.ds(base, BPW)], idx_v)
        pltpu.sync_copy(ctxv_hbm.at[pl.ds(base, BPW)], w_v)

        def issue(e, buf, sem):
            pltpu.make_async_copy(
                tab_hbm.at[idx_v.at[e, pl.ds(0, G0)]],
                buf.at[pl.ds(0, G0)], sem).start()
            pltpu.make_async_copy(
                tab_hbm.at[idx_v.at[e, pl.ds(G0, G1)]],
                buf.at[pl.ds(G0, G1)], sem).start()

        def drain(e, buf, sem):
            pltpu.make_async_copy(
                tab_hbm.at[idx_v.at[e, pl.ds(0, G0)]],
                buf.at[pl.ds(0, G0)], sem).wait()
            pltpu.make_async_copy(
                tab_hbm.at[idx_v.at[e, pl.ds(G0, G1)]],
                buf.at[pl.ds(G0, G1)], sem).wait()

        def compute(e, buf):
            def body(l0, accs):
                for u in range(UNROLL):
                    l = l0 * UNROLL + u
                    wv = plsc.load_gather(
                        w_v, [jnp.full((L,), e, jnp.int32),
                              jnp.full((L,), l, jnp.int32)])
                    accs = tuple(acc + wv * buf[l, pl.ds(j * L, L)]
                                 for j, acc in enumerate(accs))
                return accs

            accs = lax.fori_loop(
                0, H // UNROLL, body,
                tuple(jnp.zeros((L,), jnp.float32) for _ in range(D // L)))
            for j in range(D // L):
                out_v[e, pl.ds(j * L, L)] = accs[j]

        issue(0, rows0, sem0)

        @pl.loop(0, BPW // 2)
        def _(p):
            e0 = p * 2
            issue(e0 + 1, rows1, sem1)
            drain(e0, rows0, sem0)
            compute(e0, rows0)
            issue(jnp.minimum(e0 + 2, BPW - 1), rows0, sem0)
            drain(e0 + 1, rows1, sem1)
            compute(e0 + 1, rows1)

        drain(BPW - 1, rows0, sem0)

        pltpu.sync_copy(out_v, ctxsum_hbm.at[pl.ds(base, BPW)])

        pltpu.sync_copy(itm_hbm.at[pl.ds(base, BPW)], itmidx_v)
        pltpu.sync_copy(tab_hbm.at[itmidx_v], itmrows_v)
        pltpu.sync_copy(itmrows_v, itmrows_hbm.at[pl.ds(base, BPW)])

    return k(ctx, ctx_v, itm_flat, embed_w)


def _tc_tail(ctx_sum, itm_rows, z, fc1_w, fc1_b):
    def body(cs_ref, it_ref, z_ref, w_ref, b_ref, o_ref):
        p = cs_ref[...] * it_ref[...] * w_ref[:, :D] + z_ref[...] * w_ref[:, D:]
        o_ref[...] = jnp.sum(p, axis=1, keepdims=True) + b_ref[...]

    return pl.pallas_call(
        body,
        out_shape=jax.ShapeDtypeStruct((B, 1), jnp.float32),
    )(ctx_sum, itm_rows, z, fc1_w, fc1_b)


def kernel(ctx, itm, pos, ctx_v, z, embed_w, fc1_w, fc1_b):
    del pos
    itm_flat = itm.reshape(B)
    ctx_sum, itm_rows = _sc_embedding_bag(ctx, ctx_v, itm_flat, embed_w)
    return _tc_tail(ctx_sum, itm_rows, z, fc1_w, fc1_b.reshape(1, 1))


# speedup vs baseline: 1.3833x; 1.0913x over previous
import dataclasses
import functools

import jax
import jax.numpy as jnp
from jax import lax
from jax.experimental import pallas as pl
from jax.experimental.pallas import tpu as pltpu
from jax.experimental.pallas import tpu_sc as plsc

NC = 2
NS = 16
L = 16
NW = NC * NS
B = 4096
H = 200
D = 64
BPW = B // NW
G0 = 128
G1 = H - G0
UNROLL = 8


def _sc_compiler_params():
    cp = pltpu.CompilerParams()
    fields = pltpu.CompilerParams.__dataclass_fields__
    if "needs_layout_passes" in fields:
        cp = dataclasses.replace(cp, needs_layout_passes=False)
    if "use_tc_tiling_on_sc" in fields:
        cp = dataclasses.replace(cp, use_tc_tiling_on_sc=False)
    return cp


def _sc_embedding_bag(ctx, ctx_v, itm_flat, embed_w):
    mesh = plsc.VectorSubcoreMesh(core_axis_name="c", subcore_axis_name="s")

    @functools.partial(
        pl.kernel,
        out_type=[jax.ShapeDtypeStruct((B, D), jnp.float32),
                  jax.ShapeDtypeStruct((B, D), jnp.float32)],
        mesh=mesh,
        compiler_params=_sc_compiler_params(),
        scratch_types=[
            pltpu.VMEM((BPW, H), jnp.int32),
            pltpu.VMEM((BPW, H), jnp.float32),
            pltpu.VMEM((H, D), jnp.float32),
            pltpu.VMEM((H, D), jnp.float32),
            pltpu.VMEM((BPW, D), jnp.float32),
            pltpu.VMEM((BPW,), jnp.int32),
            pltpu.VMEM((BPW, D), jnp.float32),
            pltpu.SemaphoreType.DMA,
            pltpu.SemaphoreType.DMA,
        ],
    )
    def k(ctx_hbm, ctxv_hbm, itm_hbm, tab_hbm, ctxsum_hbm, itmrows_hbm,
          idx_v, w_v, rows0, rows1, out_v, itmidx_v, itmrows_v, sem0, sem1):
        wid = lax.axis_index("s") * NC + lax.axis_index("c")
        base = wid * BPW

        pltpu.sync_copy(ctx_hbm.at[pl.ds(base, BPW)], idx_v)
        pltpu.sync_copy(ctxv_hbm.at[pl.ds(base, BPW)], w_v)

        def issue(e, buf, sem):
            pltpu.make_async_copy(
                tab_hbm.at[idx_v.at[e, pl.ds(0, G0)]],
                buf.at[pl.ds(0, G0)], sem).start()
            pltpu.make_async_copy(
                tab_hbm.at[idx_v.at[e, pl.ds(G0, G1)]],
                buf.at[pl.ds(G0, G1)], sem).start()

        def drain(e, buf, sem):
            pltpu.make_async_copy(
                tab_hbm.at[idx_v.at[e, pl.ds(0, G0)]],
                buf.at[pl.ds(0, G0)], sem).wait()
            pltpu.make_async_copy(
                tab_hbm.at[idx_v.at[e, pl.ds(G0, G1)]],
                buf.at[pl.ds(G0, G1)], sem).wait()

        def compute(e, buf):
            def body(l0, accs):
                for u in range(UNROLL):
                    l = l0 * UNROLL + u
                    wv = plsc.load_gather(
                        w_v, [jnp.full((L,), e, jnp.int32),
                              jnp.full((L,), l, jnp.int32)])
                    accs = tuple(acc + wv * buf[l, pl.ds(j * L, L)]
                                 for j, acc in enumerate(accs))
                return accs

            accs = lax.fori_loop(
                0, H // UNROLL, body,
                tuple(jnp.zeros((L,), jnp.float32) for _ in range(D // L)))
            for j in range(D // L):
                out_v[e, pl.ds(j * L, L)] = accs[j]

        issue(0, rows0, sem0)

        @pl.loop(0, BPW // 2)
        def _(p):
            e0 = p * 2
            issue(e0 + 1, rows1, sem1)
            drain(e0, rows0, sem0)
            compute(e0, rows0)
            issue(jnp.minimum(e0 + 2, BPW - 1), rows0, sem0)
            drain(e0 + 1, rows1, sem1)
            compute(e0 + 1, rows1)

        drain(BPW - 1, rows0, sem0)

        pltpu.sync_copy(out_v, ctxsum_hbm.at[pl.ds(base, BPW)])

        pltpu.sync_copy(itm_hbm.at[pl.ds(base, BPW)], itmidx_v)
        pltpu.sync_copy(tab_hbm.at[itmidx_v], itmrows_v)
        pltpu.sync_copy(itmrows_v, itmrows_hbm.at[pl.ds(base, BPW)])

    return k(ctx, ctx_v, itm_flat, embed_w)


def _tc_tail(ctx_sum, itm_rows, z, fc1_w, fc1_b):
    def body(cs_ref, it_ref, z_ref, w_ref, b_ref, o_ref):
        p = cs_ref[...] * it_ref[...] * w_ref[:, :D] + z_ref[...] * w_ref[:, D:]
        o_ref[...] = jnp.sum(p, axis=1, keepdims=True) + b_ref[...]

    return pl.pallas_call(
        body,
        out_shape=jax.ShapeDtypeStruct((B, 1), jnp.float32),
    )(ctx_sum, itm_rows, z, fc1_w, fc1_b)


def kernel(ctx, itm, pos, ctx_v, z, embed_w, fc1_w, fc1_b):
    del pos
    v = embed_w.shape[0]
    tab2 = jnp.pad(embed_w, ((0, 0), (0, D))).reshape(2 * v, D)
    ctx_sum, itm_rows = _sc_embedding_bag(
        ctx * 2, ctx_v, itm.reshape(B) * 2, tab2)
    return _tc_tail(ctx_sum, itm_rows, z, fc1_w, fc1_b.reshape(1, 1))
